# baseline (speedup 1.0000x reference)
import jax
import jax.numpy as jnp
from jax import lax
from jax.experimental import pallas as pl
from jax.experimental.pallas import tpu as pltpu
from jax.experimental.pallas import tpu_sc as plsc

_N = 10000
_D = 128
_R = 4
_E = 160000

_NS = 16
_ROWS_PER_SUB = 624
_ECHUNK = 64
_NCHUNKS = _E // _ECHUNK
_DEG_CHUNK = 2000
_EPD = _E // 4

_sc_mesh = plsc.VectorSubcoreMesh(core_axis_name="c", subcore_axis_name="s")


def _degree_body(idx_hbm, out_hbm, counts_v, idx_v):
    c = lax.axis_index("c")
    s = lax.axis_index("s")
    wid = c * _NS + s
    a = wid // 4
    q = wid % 4
    zero16 = jnp.zeros((16,), jnp.float32)
    ones16 = jnp.ones((16,), jnp.float32)

    def zfill(j, carry):
        counts_v[pl.ds(j * 16, 16)] = zero16
        return carry

    lax.fori_loop(0, _N // 16, zfill, 0)

    base = a * _E + q * _EPD

    def chunk(i, carry):
        pltpu.sync_copy(idx_hbm.at[pl.ds(base + i * _DEG_CHUNK, _DEG_CHUNK)],
                        idx_v)

        def inner(j, icarry):
            v = idx_v[pl.ds(j * 16, 16)]
            plsc.addupdate_scatter(counts_v, [v], ones16)
            return icarry

        lax.fori_loop(0, _DEG_CHUNK // 16, inner, 0)
        return carry

    lax.fori_loop(0, _EPD // _DEG_CHUNK, chunk, 0)
    pltpu.sync_copy(counts_v, out_hbm.at[pl.ds(q * (8 * _N) + a * _N, _N)])


_degree_call = pl.kernel(
    _degree_body,
    out_type=jax.ShapeDtypeStruct((4 * 8 * _N,), jnp.float32),
    scratch_types=[
        pltpu.VMEM((_N,), jnp.float32),
        pltpu.VMEM((_DEG_CHUNK,), jnp.int32),
    ],
    mesh=_sc_mesh,
    compiler_params=pltpu.CompilerParams(needs_layout_passes=False),
)


_CPT = _NCHUNKS // _NS


def _edge_body(hs_hbm, rw_hbm, src_hbm, dst_hbm, out_hbm,
               agg_sp,
               rows0, rows1, rw0, rw1, prod0, prod1,
               sidx0, sidx1, didx0, didx1, sdidx0, sdidx1,
               lsem0, lsem1, ssem0, ssem1):
    c = lax.axis_index("c")
    s = lax.axis_index("s")
    zero16 = jnp.zeros((16,), jnp.float32)
    rows = (rows0, rows1)
    rwb = (rw0, rw1)
    prod = (prod0, prod1)
    sidx = (sidx0, sidx1)
    didx = (didx0, didx1)
    sdidx = (sdidx0, sdidx1)
    lsem = (lsem0, lsem1)
    ssem = (ssem0, ssem1)
    row0 = s * _ROWS_PER_SUB

    def load_idx(b, ebase):
        pltpu.sync_copy(src_hbm.at[pl.ds(ebase, _ECHUNK)], sidx[b])
        pltpu.sync_copy(dst_hbm.at[pl.ds(ebase, _ECHUNK)], didx[b])

    def start_loads(b, ebase):
        pltpu.async_copy(hs_hbm.at[sidx[b]], rows[b], lsem[b])
        pltpu.async_copy(rw_hbm.at[pl.ds(ebase, _ECHUNK)], rwb[b], lsem[b])

    def wait_loads(b, ebase):
        pltpu.make_async_copy(hs_hbm.at[sidx[b]], rows[b], lsem[b]).wait()
        pltpu.make_async_copy(rw_hbm.at[pl.ds(ebase, _ECHUNK)], rwb[b],
                              lsem[b]).wait()

    def wait_scatter(b):
        pltpu.make_async_copy(prod[b], agg_sp.at[sdidx[b]], ssem[b]).wait()

    def mul_into(b):
        for i8 in range(_ECHUNK // 16):
            sdidx[b][pl.ds(i8 * 16, 16)] = didx[b][pl.ds(i8 * 16, 16)]

        def mul(e, icarry):
            for dcol in range(8):
                sl = pl.ds(dcol * 16, 16)
                prod[b][e, sl] = rwb[b][e, sl] * rows[b][e, sl]
            return icarry

        lax.fori_loop(0, _ECHUNK, mul, 0, unroll=4)

    for j in range(2):
        r = c * 2 + j
        rel_base = r * _E

        def zfill(e, carry):
            for dcol in range(8):
                prod0[e, pl.ds(dcol * 16, 16)] = zero16
            return carry

        lax.fori_loop(0, _ECHUNK, zfill, 0)
        for kk in range(9):
            pltpu.sync_copy(prod0, agg_sp.at[pl.ds(row0 + kk * 64, 64)])

        @pl.when(s < 15)
        def _zero_tail():
            pltpu.sync_copy(prod0.at[pl.ds(0, 48)],
                            agg_sp.at[pl.ds(row0 + 576, 48)])

        @pl.when(s == 15)
        def _zero_tail15():
            pltpu.sync_copy(prod0, agg_sp.at[pl.ds(row0 + 576, 64)])

        plsc.subcore_barrier()

        chunk0 = s * _CPT
        e0 = rel_base + chunk0 * _ECHUNK
        load_idx(0, e0)
        start_loads(0, e0)
        load_idx(1, e0 + _ECHUNK)
        start_loads(1, e0 + _ECHUNK)

        def pair(t, carry):
            eA = rel_base + (chunk0 + 2 * t) * _ECHUNK
            eB = eA + _ECHUNK
            wait_loads(0, eA)

            @pl.when(t > 0)
            def _drain_a():
                wait_scatter(0)

            mul_into(0)
            pltpu.async_copy(prod0, agg_sp.at[sdidx0], ssem0, add=True)

            @pl.when(t < _CPT // 2 - 1)
            def _next_a():
                load_idx(0, eA + 2 * _ECHUNK)
                start_loads(0, eA + 2 * _ECHUNK)

            wait_loads(1, eB)

            @pl.when(t > 0)
            def _drain_b():
                wait_scatter(1)

            mul_into(1)
            pltpu.async_copy(prod1, agg_sp.at[sdidx1], ssem1, add=True)

            @pl.when(t < _CPT // 2 - 1)
            def _next_b():
                load_idx(1, eB + 2 * _ECHUNK)
                start_loads(1, eB + 2 * _ECHUNK)

            return carry

        lax.fori_loop(0, _CPT // 2, pair, 0)
        wait_scatter(0)
        wait_scatter(1)

        @pl.when(s < 4)
        def _leftover():
            eL = rel_base + (_NS * _CPT + s) * _ECHUNK
            load_idx(0, eL)
            start_loads(0, eL)
            wait_loads(0, eL)
            mul_into(0)
            pltpu.sync_copy(prod0, agg_sp.at[sdidx0], add=True)

        plsc.subcore_barrier()

        out_base = r * _N + row0
        for kk in range(4):
            pltpu.sync_copy(agg_sp.at[pl.ds(row0 + kk * 128, 128)],
                            out_hbm.at[pl.ds(out_base + kk * 128, 128)])

        @pl.when(s < 15)
        def _dump_tail():
            pltpu.sync_copy(agg_sp.at[pl.ds(row0 + 512, 112)],
                            out_hbm.at[pl.ds(out_base + 512, 112)])

        @pl.when(s == 15)
        def _dump_tail15():
            pltpu.sync_copy(agg_sp.at[pl.ds(row0 + 512, 128)],
                            out_hbm.at[pl.ds(out_base + 512, 128)])

        plsc.subcore_barrier()


_edge_call = pl.kernel(
    _edge_body,
    out_type=jax.ShapeDtypeStruct((4 * _N, _D), jnp.float32),
    scratch_types=(
        [pltpu.VMEM_SHARED((_N, _D), jnp.float32)]
        + [pltpu.VMEM((_ECHUNK, _D), jnp.float32)] * 6
        + [pltpu.VMEM((_ECHUNK,), jnp.int32)] * 6
        + [pltpu.SemaphoreType.DMA] * 4
    ),
    mesh=_sc_mesh,
    compiler_params=pltpu.CompilerParams(needs_layout_passes=False),
)


_BN = 1024


def _prescale_body(feat_ref, wt_ref, b_ref, cnt_ref, hs_ref, s_ref):
    h = jnp.dot(feat_ref[...], wt_ref[...],
                preferred_element_type=jnp.float32) + b_ref[...]
    deg = jnp.sum(cnt_ref[...], axis=0)
    sfac = lax.rsqrt(jnp.maximum(deg, 1.0))
    s_ref[...] = sfac
    for r in range(_R):
        hs_ref[r] = h * sfac[r][:, None]


_prescale_call = pl.pallas_call(
    _prescale_body,
    grid=(pl.cdiv(_N, _BN),),
    in_specs=[
        pl.BlockSpec((_BN, _D), lambda i: (i, 0)),
        pl.BlockSpec((_D, _D), lambda i: (0, 0)),
        pl.BlockSpec((1, _D), lambda i: (0, 0)),
        pl.BlockSpec((4, 8, _BN), lambda i: (0, 0, i)),
    ],
    out_specs=[
        pl.BlockSpec((_R, _BN, _D), lambda i: (0, i, 0)),
        pl.BlockSpec((8, _BN), lambda i: (0, i)),
    ],
    out_shape=[
        jax.ShapeDtypeStruct((_R, _N, _D), jnp.float32),
        jax.ShapeDtypeStruct((8, _N), jnp.float32),
    ],
)

_BE = 2048


def _rw_body(ew_ref, wt_ref, b_ref, out_ref):
    out_ref[...] = jnp.dot(ew_ref[...], wt_ref[...],
                           preferred_element_type=jnp.float32) + b_ref[...]


_rw_call = pl.pallas_call(
    _rw_body,
    grid=(pl.cdiv(_R * _E, _BE),),
    in_specs=[
        pl.BlockSpec((_BE, _D), lambda i: (i, 0)),
        pl.BlockSpec((_D, _D), lambda i: (0, 0)),
        pl.BlockSpec((1, _D), lambda i: (0, 0)),
    ],
    out_specs=pl.BlockSpec((_BE, _D), lambda i: (i, 0)),
    out_shape=jax.ShapeDtypeStruct((_R * _E, _D), jnp.float32),
)


def _combine1_body(agg_ref, w_ref, b_ref, s_ref, hs2_ref):
    acc = None
    for r in range(_R):
        t = jnp.dot(agg_ref[r], w_ref[r], preferred_element_type=jnp.float32)
        t = t * s_ref[4 + r][:, None] + b_ref[r][None, :]
        acc = t if acc is None else acc + t
    emb0 = acc * 0.25
    for r in range(_R):
        hs2_ref[r] = emb0 * s_ref[r][:, None]


_combine1_call = pl.pallas_call(
    _combine1_body,
    grid=(pl.cdiv(_N, _BN),),
    in_specs=[
        pl.BlockSpec((_R, _BN, _D), lambda i: (0, i, 0)),
        pl.BlockSpec((_R, _D, _D), lambda i: (0, 0, 0)),
        pl.BlockSpec((_R, _D), lambda i: (0, 0)),
        pl.BlockSpec((8, _BN), lambda i: (0, i)),
    ],
    out_specs=pl.BlockSpec((_R, _BN, _D), lambda i: (0, i, 0)),
    out_shape=jax.ShapeDtypeStruct((_R, _N, _D), jnp.float32),
)


def _combine2_body(agg_ref, w_ref, b_ref, s_ref, out_ref):
    acc = None
    for r in range(_R):
        t = jnp.dot(agg_ref[r], w_ref[r], preferred_element_type=jnp.float32)
        t = t * s_ref[4 + r][:, None] + b_ref[r][None, :]
        acc = t if acc is None else acc + t
    out_ref[...] = acc * 0.25


_combine2_call = pl.pallas_call(
    _combine2_body,
    grid=(pl.cdiv(_N, _BN),),
    in_specs=[
        pl.BlockSpec((_R, _BN, _D), lambda i: (0, i, 0)),
        pl.BlockSpec((_R, _D, _D), lambda i: (0, 0, 0)),
        pl.BlockSpec((_R, _D), lambda i: (0, 0)),
        pl.BlockSpec((8, _BN), lambda i: (0, i)),
    ],
    out_specs=pl.BlockSpec((_BN, _D), lambda i: (i, 0)),
    out_shape=jax.ShapeDtypeStruct((_N, _D), jnp.float32),
)


def kernel(feat, edge_index_r0, edge_index_r1, edge_index_r2, edge_index_r3,
           edge_weight_r0, edge_weight_r1, edge_weight_r2, edge_weight_r3,
           node_fc_W, node_fc_b, rela_fc_W, rela_fc_b, W1, b1, W2, b2):
    eis = [edge_index_r0, edge_index_r1, edge_index_r2, edge_index_r3]
    src = jnp.stack([e[0] for e in eis]).astype(jnp.int32)
    dst = jnp.stack([e[1] for e in eis]).astype(jnp.int32)
    idx_all = jnp.concatenate([src, dst], axis=0).reshape(-1)

    counts = _degree_call(idx_all).reshape(4, 8, _N)
    hs, s_all = _prescale_call(feat, node_fc_W.T, node_fc_b[None, :], counts)

    ew_all = jnp.concatenate(
        [edge_weight_r0, edge_weight_r1, edge_weight_r2, edge_weight_r3], axis=0)
    rw = _rw_call(ew_all, rela_fc_W.T, rela_fc_b[None, :])

    src_pre = (src + (jnp.arange(_R, dtype=jnp.int32) * _N)[:, None]).reshape(-1)
    dst_flat = dst.reshape(-1)

    agg1 = _edge_call(hs.reshape(_R * _N, _D), rw, src_pre, dst_flat)
    hs2 = _combine1_call(agg1.reshape(_R, _N, _D), W1, b1, s_all)
    agg2 = _edge_call(hs2.reshape(_R * _N, _D), rw, src_pre, dst_flat)
    return _combine2_call(agg2.reshape(_R, _N, _D), W2, b2, s_all)


# speedup vs baseline: 1.7061x; 1.7061x over previous
import jax
import jax.numpy as jnp
from jax import lax
from jax.experimental import pallas as pl
from jax.experimental.pallas import tpu as pltpu
from jax.experimental.pallas import tpu_sc as plsc

_N = 10000
_D = 128
_R = 4
_E = 160000

_NS = 16
_ROWS_PER_SUB = 624
_ECHUNK = 64
_NCHUNKS = _E // _ECHUNK
_DEG_CHUNK = 2000
_EPD = _E // 4

_sc_mesh = plsc.VectorSubcoreMesh(core_axis_name="c", subcore_axis_name="s")


def _degree_body(idx_hbm, out_hbm, counts_v, idx_v):
    c = lax.axis_index("c")
    s = lax.axis_index("s")
    wid = c * _NS + s
    a = wid // 4
    q = wid % 4
    zero16 = jnp.zeros((16,), jnp.float32)
    ones16 = jnp.ones((16,), jnp.float32)

    def zfill(j, carry):
        counts_v[pl.ds(j * 16, 16)] = zero16
        return carry

    lax.fori_loop(0, _N // 16, zfill, 0)

    base = a * _E + q * _EPD

    def chunk(i, carry):
        pltpu.sync_copy(idx_hbm.at[pl.ds(base + i * _DEG_CHUNK, _DEG_CHUNK)],
                        idx_v)

        def inner(j, icarry):
            v = idx_v[pl.ds(j * 16, 16)]
            plsc.addupdate_scatter(counts_v, [v], ones16)
            return icarry

        lax.fori_loop(0, _DEG_CHUNK // 16, inner, 0)
        return carry

    lax.fori_loop(0, _EPD // _DEG_CHUNK, chunk, 0)
    pltpu.sync_copy(counts_v, out_hbm.at[pl.ds(q * (8 * _N) + a * _N, _N)])


_degree_call = pl.kernel(
    _degree_body,
    out_type=jax.ShapeDtypeStruct((4 * 8 * _N,), jnp.float32),
    scratch_types=[
        pltpu.VMEM((_N,), jnp.float32),
        pltpu.VMEM((_DEG_CHUNK,), jnp.int32),
    ],
    mesh=_sc_mesh,
    compiler_params=pltpu.CompilerParams(needs_layout_passes=False),
)


_CPT = _NCHUNKS // _NS


def _edge_body(hs_hbm, rw_hbm, src_hbm, dst_hbm, out_hbm,
               agg_sp,
               rows0, rows1, rows2, rw0, rw1, rw2,
               sidx0, sidx1, sidx2, didx0, didx1, didx2,
               sdidx0, sdidx1, sdidx2,
               lsem0, lsem1, lsem2, ssem0, ssem1, ssem2):
    c = lax.axis_index("c")
    s = lax.axis_index("s")
    zero16 = jnp.zeros((16,), jnp.float32)
    rows = (rows0, rows1, rows2)
    rwb = (rw0, rw1, rw2)
    sidx = (sidx0, sidx1, sidx2)
    didx = (didx0, didx1, didx2)
    sdidx = (sdidx0, sdidx1, sdidx2)
    lsem = (lsem0, lsem1, lsem2)
    ssem = (ssem0, ssem1, ssem2)
    row0 = s * _ROWS_PER_SUB
    _STEPS = _CPT // 3

    def load_idx(b, ebase):
        pltpu.sync_copy(src_hbm.at[pl.ds(ebase, _ECHUNK)], sidx[b])
        pltpu.sync_copy(dst_hbm.at[pl.ds(ebase, _ECHUNK)], didx[b])

    def start_loads(b, ebase):
        pltpu.async_copy(hs_hbm.at[sidx[b]], rows[b], lsem[b])
        pltpu.async_copy(rw_hbm.at[pl.ds(ebase, _ECHUNK)], rwb[b], lsem[b])

    def wait_loads(b, ebase):
        pltpu.make_async_copy(hs_hbm.at[sidx[b]], rows[b], lsem[b]).wait()
        pltpu.make_async_copy(rw_hbm.at[pl.ds(ebase, _ECHUNK)], rwb[b],
                              lsem[b]).wait()

    def wait_scatter(b):
        pltpu.make_async_copy(rwb[b], agg_sp.at[sdidx[b]], ssem[b]).wait()

    def mul_into(b):
        for i8 in range(_ECHUNK // 16):
            sdidx[b][pl.ds(i8 * 16, 16)] = didx[b][pl.ds(i8 * 16, 16)]

        def mul(e, icarry):
            for dcol in range(8):
                sl = pl.ds(dcol * 16, 16)
                rwb[b][e, sl] = rwb[b][e, sl] * rows[b][e, sl]
            return icarry

        lax.fori_loop(0, _ECHUNK, mul, 0)

    for j in range(2):
        r = c * 2 + j
        rel_base = r * _E

        def zfill(e, carry):
            for dcol in range(8):
                rw0[e, pl.ds(dcol * 16, 16)] = zero16
            return carry

        lax.fori_loop(0, _ECHUNK, zfill, 0)
        for kk in range(9):
            pltpu.sync_copy(rw0, agg_sp.at[pl.ds(row0 + kk * 64, 64)])

        @pl.when(s < 15)
        def _zero_tail():
            pltpu.sync_copy(rw0.at[pl.ds(0, 48)],
                            agg_sp.at[pl.ds(row0 + 576, 48)])

        @pl.when(s == 15)
        def _zero_tail15():
            pltpu.sync_copy(rw0, agg_sp.at[pl.ds(row0 + 576, 64)])

        plsc.subcore_barrier()

        chunk0 = s * _CPT
        e0 = rel_base + chunk0 * _ECHUNK
        load_idx(0, e0)
        start_loads(0, e0)
        load_idx(1, e0 + _ECHUNK)
        start_loads(1, e0 + _ECHUNK)

        def triple(t, carry):
            for k in range(3):
                me = rel_base + (chunk0 + 3 * t + k) * _ECHUNK
                wait_loads(k, me)
                mul_into(k)
                pltpu.async_copy(rwb[k], agg_sp.at[sdidx[k]], ssem[k],
                                 add=True)
                kp = (k + 2) % 3
                pe = me + 2 * _ECHUNK
                if k == 0:
                    @pl.when(t > 0)
                    def _drain0():
                        wait_scatter(kp)

                    load_idx(kp, pe)
                    start_loads(kp, pe)
                else:
                    @pl.when(t < _STEPS - 1)
                    def _prep():
                        wait_scatter(kp)
                        load_idx(kp, pe)
                        start_loads(kp, pe)
            return carry

        lax.fori_loop(0, _STEPS, triple, 0)
        wait_scatter(0)
        wait_scatter(1)
        wait_scatter(2)

        @pl.when(s < 4)
        def _leftover():
            eL = rel_base + (_NS * _CPT + s) * _ECHUNK
            load_idx(0, eL)
            start_loads(0, eL)
            wait_loads(0, eL)
            mul_into(0)
            pltpu.sync_copy(rw0, agg_sp.at[sdidx0], add=True)

        plsc.subcore_barrier()

        out_base = r * _N + row0
        for kk in range(4):
            pltpu.sync_copy(agg_sp.at[pl.ds(row0 + kk * 128, 128)],
                            out_hbm.at[pl.ds(out_base + kk * 128, 128)])

        @pl.when(s < 15)
        def _dump_tail():
            pltpu.sync_copy(agg_sp.at[pl.ds(row0 + 512, 112)],
                            out_hbm.at[pl.ds(out_base + 512, 112)])

        @pl.when(s == 15)
        def _dump_tail15():
            pltpu.sync_copy(agg_sp.at[pl.ds(row0 + 512, 128)],
                            out_hbm.at[pl.ds(out_base + 512, 128)])

        plsc.subcore_barrier()


_edge_call = pl.kernel(
    _edge_body,
    out_type=jax.ShapeDtypeStruct((4 * _N, _D), jnp.float32),
    scratch_types=(
        [pltpu.VMEM_SHARED((_N, _D), jnp.float32)]
        + [pltpu.VMEM((_ECHUNK, _D), jnp.float32)] * 6
        + [pltpu.VMEM((_ECHUNK,), jnp.int32)] * 9
        + [pltpu.SemaphoreType.DMA] * 6
    ),
    mesh=_sc_mesh,
    compiler_params=pltpu.CompilerParams(needs_layout_passes=False),
)


_BN = 1024


def _prescale_body(feat_ref, wt_ref, b_ref, cnt_ref, hs_ref, s_ref):
    h = jnp.dot(feat_ref[...], wt_ref[...],
                preferred_element_type=jnp.float32) + b_ref[...]
    deg = jnp.sum(cnt_ref[...], axis=0)
    sfac = lax.rsqrt(jnp.maximum(deg, 1.0))
    s_ref[...] = sfac
    for r in range(_R):
        hs_ref[r] = h * sfac[r][:, None]


_prescale_call = pl.pallas_call(
    _prescale_body,
    grid=(pl.cdiv(_N, _BN),),
    in_specs=[
        pl.BlockSpec((_BN, _D), lambda i: (i, 0)),
        pl.BlockSpec((_D, _D), lambda i: (0, 0)),
        pl.BlockSpec((1, _D), lambda i: (0, 0)),
        pl.BlockSpec((4, 8, _BN), lambda i: (0, 0, i)),
    ],
    out_specs=[
        pl.BlockSpec((_R, _BN, _D), lambda i: (0, i, 0)),
        pl.BlockSpec((8, _BN), lambda i: (0, i)),
    ],
    out_shape=[
        jax.ShapeDtypeStruct((_R, _N, _D), jnp.float32),
        jax.ShapeDtypeStruct((8, _N), jnp.float32),
    ],
)

_BE = 2048


def _rw_body(ew_ref, wt_ref, b_ref, out_ref):
    out_ref[...] = jnp.dot(ew_ref[...], wt_ref[...],
                           preferred_element_type=jnp.float32) + b_ref[...]


_rw_call = pl.pallas_call(
    _rw_body,
    grid=(pl.cdiv(_R * _E, _BE),),
    in_specs=[
        pl.BlockSpec((_BE, _D), lambda i: (i, 0)),
        pl.BlockSpec((_D, _D), lambda i: (0, 0)),
        pl.BlockSpec((1, _D), lambda i: (0, 0)),
    ],
    out_specs=pl.BlockSpec((_BE, _D), lambda i: (i, 0)),
    out_shape=jax.ShapeDtypeStruct((_R * _E, _D), jnp.float32),
)


def _combine1_body(agg_ref, w_ref, b_ref, s_ref, hs2_ref):
    acc = None
    for r in range(_R):
        t = jnp.dot(agg_ref[r], w_ref[r], preferred_element_type=jnp.float32)
        t = t * s_ref[4 + r][:, None] + b_ref[r][None, :]
        acc = t if acc is None else acc + t
    emb0 = acc * 0.25
    for r in range(_R):
        hs2_ref[r] = emb0 * s_ref[r][:, None]


_combine1_call = pl.pallas_call(
    _combine1_body,
    grid=(pl.cdiv(_N, _BN),),
    in_specs=[
        pl.BlockSpec((_R, _BN, _D), lambda i: (0, i, 0)),
        pl.BlockSpec((_R, _D, _D), lambda i: (0, 0, 0)),
        pl.BlockSpec((_R, _D), lambda i: (0, 0)),
        pl.BlockSpec((8, _BN), lambda i: (0, i)),
    ],
    out_specs=pl.BlockSpec((_R, _BN, _D), lambda i: (0, i, 0)),
    out_shape=jax.ShapeDtypeStruct((_R, _N, _D), jnp.float32),
)


def _combine2_body(agg_ref, w_ref, b_ref, s_ref, out_ref):
    acc = None
    for r in range(_R):
        t = jnp.dot(agg_ref[r], w_ref[r], preferred_element_type=jnp.float32)
        t = t * s_ref[4 + r][:, None] + b_ref[r][None, :]
        acc = t if acc is None else acc + t
    out_ref[...] = acc * 0.25


_combine2_call = pl.pallas_call(
    _combine2_body,
    grid=(pl.cdiv(_N, _BN),),
    in_specs=[
        pl.BlockSpec((_R, _BN, _D), lambda i: (0, i, 0)),
        pl.BlockSpec((_R, _D, _D), lambda i: (0, 0, 0)),
        pl.BlockSpec((_R, _D), lambda i: (0, 0)),
        pl.BlockSpec((8, _BN), lambda i: (0, i)),
    ],
    out_specs=pl.BlockSpec((_BN, _D), lambda i: (i, 0)),
    out_shape=jax.ShapeDtypeStruct((_N, _D), jnp.float32),
)


def kernel(feat, edge_index_r0, edge_index_r1, edge_index_r2, edge_index_r3,
           edge_weight_r0, edge_weight_r1, edge_weight_r2, edge_weight_r3,
           node_fc_W, node_fc_b, rela_fc_W, rela_fc_b, W1, b1, W2, b2):
    eis = [edge_index_r0, edge_index_r1, edge_index_r2, edge_index_r3]
    src = jnp.stack([e[0] for e in eis]).astype(jnp.int32)
    dst = jnp.stack([e[1] for e in eis]).astype(jnp.int32)
    idx_all = jnp.concatenate([src, dst], axis=0).reshape(-1)

    counts = _degree_call(idx_all).reshape(4, 8, _N)
    hs, s_all = _prescale_call(feat, node_fc_W.T, node_fc_b[None, :], counts)

    ew_all = jnp.concatenate(
        [edge_weight_r0, edge_weight_r1, edge_weight_r2, edge_weight_r3], axis=0)
    rw = _rw_call(ew_all, rela_fc_W.T, rela_fc_b[None, :])

    src_pre = (src + (jnp.arange(_R, dtype=jnp.int32) * _N)[:, None]).reshape(-1)
    dst_flat = dst.reshape(-1)

    agg1 = _edge_call(hs.reshape(_R * _N, _D), rw, src_pre, dst_flat)
    hs2 = _combine1_call(agg1.reshape(_R, _N, _D), W1, b1, s_all)
    agg2 = _edge_call(hs2.reshape(_R * _N, _D), rw, src_pre, dst_flat)
    return _combine2_call(agg2.reshape(_R, _N, _D), W2, b2, s_all)


# speedup vs baseline: 1.9876x; 1.1650x over previous
import jax
import jax.numpy as jnp
from jax import lax
from jax.experimental import pallas as pl
from jax.experimental.pallas import tpu as pltpu
from jax.experimental.pallas import tpu_sc as plsc

_N = 10000
_D = 128
_R = 4
_E = 160000

_NS = 16
_ROWS_PER_SUB = 624
_ECHUNK = 64
_NCHUNKS = _E // _ECHUNK
_DEG_CHUNK = 2000
_EPD = _E // 4

_sc_mesh = plsc.VectorSubcoreMesh(core_axis_name="c", subcore_axis_name="s")


def _degree_body(idx_hbm, out_hbm, counts_v, idx_v):
    c = lax.axis_index("c")
    s = lax.axis_index("s")
    wid = c * _NS + s
    a = wid // 4
    q = wid % 4
    zero16 = jnp.zeros((16,), jnp.float32)
    ones16 = jnp.ones((16,), jnp.float32)

    def zfill(j, carry):
        counts_v[pl.ds(j * 16, 16)] = zero16
        return carry

    lax.fori_loop(0, _N // 16, zfill, 0)

    base = a * _E + q * _EPD

    def chunk(i, carry):
        pltpu.sync_copy(idx_hbm.at[pl.ds(base + i * _DEG_CHUNK, _DEG_CHUNK)],
                        idx_v)

        def inner(j, icarry):
            v = idx_v[pl.ds(j * 16, 16)]
            plsc.addupdate_scatter(counts_v, [v], ones16)
            return icarry

        lax.fori_loop(0, _DEG_CHUNK // 16, inner, 0)
        return carry

    lax.fori_loop(0, _EPD // _DEG_CHUNK, chunk, 0)
    pltpu.sync_copy(counts_v, out_hbm.at[pl.ds(q * (8 * _N) + a * _N, _N)])


_degree_call = pl.kernel(
    _degree_body,
    out_type=jax.ShapeDtypeStruct((4 * 8 * _N,), jnp.float32),
    scratch_types=[
        pltpu.VMEM((_N,), jnp.float32),
        pltpu.VMEM((_DEG_CHUNK,), jnp.int32),
    ],
    mesh=_sc_mesh,
    compiler_params=pltpu.CompilerParams(needs_layout_passes=False),
)


_CPT = _NCHUNKS // _NS


def _edge_body(hs_hbm, rw_hbm, src_hbm, dst_hbm, out_hbm,
               agg_sp,
               rows0, rows1, rows2, rw0, rw1, rw2,
               sblk0, sblk1, dblk0, dblk1,
               sdidx0, sdidx1, sdidx2,
               lsem0, lsem1, lsem2, ssem0, ssem1, ssem2):
    c = lax.axis_index("c")
    s = lax.axis_index("s")
    zero16 = jnp.zeros((16,), jnp.float32)
    rows = (rows0, rows1, rows2)
    rwb = (rw0, rw1, rw2)
    sblk = (sblk0, sblk1)
    dblk = (dblk0, dblk1)
    sdidx = (sdidx0, sdidx1, sdidx2)
    lsem = (lsem0, lsem1, lsem2)
    ssem = (ssem0, ssem1, ssem2)
    row0 = s * _ROWS_PER_SUB
    _BLK = 6
    _NBLK = _CPT // _BLK

    def start_loads(b, p, off, ebase):
        pltpu.async_copy(hs_hbm.at[sblk[p].at[pl.ds(off, _ECHUNK)]],
                         rows[b], lsem[b])
        pltpu.async_copy(rw_hbm.at[pl.ds(ebase, _ECHUNK)], rwb[b], lsem[b])

    def wait_loads(b, p, off, ebase):
        pltpu.make_async_copy(hs_hbm.at[sblk[p].at[pl.ds(off, _ECHUNK)]],
                              rows[b], lsem[b]).wait()
        pltpu.make_async_copy(rw_hbm.at[pl.ds(ebase, _ECHUNK)], rwb[b],
                              lsem[b]).wait()

    def wait_scatter(b):
        pltpu.make_async_copy(rwb[b], agg_sp.at[sdidx[b]], ssem[b]).wait()

    def mul_into(b, p, off):
        for i8 in range(_ECHUNK // 16):
            sdidx[b][pl.ds(i8 * 16, 16)] = dblk[p][pl.ds(off + i8 * 16, 16)]

        def mul(e, icarry):
            for dcol in range(8):
                sl = pl.ds(dcol * 16, 16)
                rwb[b][e, sl] = rwb[b][e, sl] * rows[b][e, sl]
            return icarry

        lax.fori_loop(0, _ECHUNK, mul, 0)

    for j in range(2):
        r = c * 2 + j
        rel_base = r * _E

        def zfill(e, carry):
            for dcol in range(8):
                rw0[e, pl.ds(dcol * 16, 16)] = zero16
            return carry

        lax.fori_loop(0, _ECHUNK, zfill, 0)
        for kk in range(9):
            pltpu.sync_copy(rw0, agg_sp.at[pl.ds(row0 + kk * 64, 64)])

        @pl.when(s < 15)
        def _zero_tail():
            pltpu.sync_copy(rw0.at[pl.ds(0, 48)],
                            agg_sp.at[pl.ds(row0 + 576, 48)])

        @pl.when(s == 15)
        def _zero_tail15():
            pltpu.sync_copy(rw0, agg_sp.at[pl.ds(row0 + 576, 64)])

        plsc.subcore_barrier()

        chunk0 = s * _CPT
        e0 = rel_base + chunk0 * _ECHUNK
        blk_edges = _BLK * _ECHUNK
        pltpu.sync_copy(src_hbm.at[pl.ds(e0, blk_edges)], sblk0)
        pltpu.sync_copy(dst_hbm.at[pl.ds(e0, blk_edges)], dblk0)
        start_loads(0, 0, 0, e0)
        start_loads(1, 0, _ECHUNK, e0 + _ECHUNK)

        def block_pair(t, carry):
            for pb in range(2):
                g = 2 * t + pb
                gbase = e0 + g * blk_edges

                @pl.when(g < _NBLK - 1)
                def _prefetch_idx():
                    pltpu.sync_copy(
                        src_hbm.at[pl.ds(gbase + blk_edges, blk_edges)],
                        sblk[1 - pb])
                    pltpu.sync_copy(
                        dst_hbm.at[pl.ds(gbase + blk_edges, blk_edges)],
                        dblk[1 - pb])

                for tt in range(2):
                    for k in range(3):
                        off = (3 * tt + k) * _ECHUNK
                        me = gbase + off
                        wait_loads(k, pb, off, me)
                        mul_into(k, pb, off)
                        pltpu.async_copy(rwb[k], agg_sp.at[sdidx[k]],
                                         ssem[k], add=True)
                        kp = (k + 2) % 3
                        poff = off + 2 * _ECHUNK
                        pe = me + 2 * _ECHUNK
                        pp = pb if poff < blk_edges else 1 - pb
                        poff2 = poff if poff < blk_edges else poff - blk_edges
                        if tt == 0 and k == 0:
                            @pl.when(g > 0)
                            def _drain_first():
                                wait_scatter(kp)

                            start_loads(kp, pp, poff2, pe)
                        elif tt == 1 and k >= 1:
                            @pl.when(g < _NBLK - 1)
                            def _prep_tail():
                                wait_scatter(kp)
                                start_loads(kp, pp, poff2, pe)
                        else:
                            wait_scatter(kp)
                            start_loads(kp, pp, poff2, pe)
            return carry

        lax.fori_loop(0, _NBLK // 2, block_pair, 0)
        wait_scatter(0)
        wait_scatter(1)
        wait_scatter(2)

        @pl.when(s < 4)
        def _leftover():
            eL = rel_base + (_NS * _CPT + s) * _ECHUNK
            pltpu.sync_copy(src_hbm.at[pl.ds(eL, _ECHUNK)],
                            sblk0.at[pl.ds(0, _ECHUNK)])
            pltpu.sync_copy(dst_hbm.at[pl.ds(eL, _ECHUNK)],
                            dblk0.at[pl.ds(0, _ECHUNK)])
            start_loads(0, 0, 0, eL)
            wait_loads(0, 0, 0, eL)
            mul_into(0, 0, 0)
            pltpu.sync_copy(rw0, agg_sp.at[sdidx0], add=True)

        plsc.subcore_barrier()

        out_base = r * _N + row0
        for kk in range(4):
            pltpu.sync_copy(agg_sp.at[pl.ds(row0 + kk * 128, 128)],
                            out_hbm.at[pl.ds(out_base + kk * 128, 128)])

        @pl.when(s < 15)
        def _dump_tail():
            pltpu.sync_copy(agg_sp.at[pl.ds(row0 + 512, 112)],
                            out_hbm.at[pl.ds(out_base + 512, 112)])

        @pl.when(s == 15)
        def _dump_tail15():
            pltpu.sync_copy(agg_sp.at[pl.ds(row0 + 512, 128)],
                            out_hbm.at[pl.ds(out_base + 512, 128)])

        plsc.subcore_barrier()


_edge_call = pl.kernel(
    _edge_body,
    out_type=jax.ShapeDtypeStruct((4 * _N, _D), jnp.float32),
    scratch_types=(
        [pltpu.VMEM_SHARED((_N, _D), jnp.float32)]
        + [pltpu.VMEM((_ECHUNK, _D), jnp.float32)] * 6
        + [pltpu.VMEM((6 * _ECHUNK,), jnp.int32)] * 4
        + [pltpu.VMEM((_ECHUNK,), jnp.int32)] * 3
        + [pltpu.SemaphoreType.DMA] * 6
    ),
    mesh=_sc_mesh,
    compiler_params=pltpu.CompilerParams(needs_layout_passes=False),
)


_BN = 1024


def _prescale_body(feat_ref, wt_ref, b_ref, cnt_ref, hs_ref, s_ref):
    h = jnp.dot(feat_ref[...], wt_ref[...],
                preferred_element_type=jnp.float32) + b_ref[...]
    deg = jnp.sum(cnt_ref[...], axis=0)
    sfac = lax.rsqrt(jnp.maximum(deg, 1.0))
    s_ref[...] = sfac
    for r in range(_R):
        hs_ref[r] = h * sfac[r][:, None]


_prescale_call = pl.pallas_call(
    _prescale_body,
    grid=(pl.cdiv(_N, _BN),),
    in_specs=[
        pl.BlockSpec((_BN, _D), lambda i: (i, 0)),
        pl.BlockSpec((_D, _D), lambda i: (0, 0)),
        pl.BlockSpec((1, _D), lambda i: (0, 0)),
        pl.BlockSpec((4, 8, _BN), lambda i: (0, 0, i)),
    ],
    out_specs=[
        pl.BlockSpec((_R, _BN, _D), lambda i: (0, i, 0)),
        pl.BlockSpec((8, _BN), lambda i: (0, i)),
    ],
    out_shape=[
        jax.ShapeDtypeStruct((_R, _N, _D), jnp.float32),
        jax.ShapeDtypeStruct((8, _N), jnp.float32),
    ],
)

_BE = 2048


def _rw_body(ew_ref, wt_ref, b_ref, out_ref):
    out_ref[...] = jnp.dot(ew_ref[...], wt_ref[...],
                           preferred_element_type=jnp.float32) + b_ref[...]


_rw_call = pl.pallas_call(
    _rw_body,
    grid=(pl.cdiv(_R * _E, _BE),),
    in_specs=[
        pl.BlockSpec((_BE, _D), lambda i: (i, 0)),
        pl.BlockSpec((_D, _D), lambda i: (0, 0)),
        pl.BlockSpec((1, _D), lambda i: (0, 0)),
    ],
    out_specs=pl.BlockSpec((_BE, _D), lambda i: (i, 0)),
    out_shape=jax.ShapeDtypeStruct((_R * _E, _D), jnp.float32),
)


def _combine1_body(agg_ref, w_ref, b_ref, s_ref, hs2_ref):
    acc = None
    for r in range(_R):
        t = jnp.dot(agg_ref[r], w_ref[r], preferred_element_type=jnp.float32)
        t = t * s_ref[4 + r][:, None] + b_ref[r][None, :]
        acc = t if acc is None else acc + t
    emb0 = acc * 0.25
    for r in range(_R):
        hs2_ref[r] = emb0 * s_ref[r][:, None]


_combine1_call = pl.pallas_call(
    _combine1_body,
    grid=(pl.cdiv(_N, _BN),),
    in_specs=[
        pl.BlockSpec((_R, _BN, _D), lambda i: (0, i, 0)),
        pl.BlockSpec((_R, _D, _D), lambda i: (0, 0, 0)),
        pl.BlockSpec((_R, _D), lambda i: (0, 0)),
        pl.BlockSpec((8, _BN), lambda i: (0, i)),
    ],
    out_specs=pl.BlockSpec((_R, _BN, _D), lambda i: (0, i, 0)),
    out_shape=jax.ShapeDtypeStruct((_R, _N, _D), jnp.float32),
)


def _combine2_body(agg_ref, w_ref, b_ref, s_ref, out_ref):
    acc = None
    for r in range(_R):
        t = jnp.dot(agg_ref[r], w_ref[r], preferred_element_type=jnp.float32)
        t = t * s_ref[4 + r][:, None] + b_ref[r][None, :]
        acc = t if acc is None else acc + t
    out_ref[...] = acc * 0.25


_combine2_call = pl.pallas_call(
    _combine2_body,
    grid=(pl.cdiv(_N, _BN),),
    in_specs=[
        pl.BlockSpec((_R, _BN, _D), lambda i: (0, i, 0)),
        pl.BlockSpec((_R, _D, _D), lambda i: (0, 0, 0)),
        pl.BlockSpec((_R, _D), lambda i: (0, 0)),
        pl.BlockSpec((8, _BN), lambda i: (0, i)),
    ],
    out_specs=pl.BlockSpec((_BN, _D), lambda i: (i, 0)),
    out_shape=jax.ShapeDtypeStruct((_N, _D), jnp.float32),
)


def kernel(feat, edge_index_r0, edge_index_r1, edge_index_r2, edge_index_r3,
           edge_weight_r0, edge_weight_r1, edge_weight_r2, edge_weight_r3,
           node_fc_W, node_fc_b, rela_fc_W, rela_fc_b, W1, b1, W2, b2):
    eis = [edge_index_r0, edge_index_r1, edge_index_r2, edge_index_r3]
    src = jnp.stack([e[0] for e in eis]).astype(jnp.int32)
    dst = jnp.stack([e[1] for e in eis]).astype(jnp.int32)
    idx_all = jnp.concatenate([src, dst], axis=0).reshape(-1)

    counts = _degree_call(idx_all).reshape(4, 8, _N)
    hs, s_all = _prescale_call(feat, node_fc_W.T, node_fc_b[None, :], counts)

    ew_all = jnp.concatenate(
        [edge_weight_r0, edge_weight_r1, edge_weight_r2, edge_weight_r3], axis=0)
    rw = _rw_call(ew_all, rela_fc_W.T, rela_fc_b[None, :])

    src_pre = (src + (jnp.arange(_R, dtype=jnp.int32) * _N)[:, None]).reshape(-1)
    dst_flat = dst.reshape(-1)

    agg1 = _edge_call(hs.reshape(_R * _N, _D), rw, src_pre, dst_flat)
    hs2 = _combine1_call(agg1.reshape(_R, _N, _D), W1, b1, s_all)
    agg2 = _edge_call(hs2.reshape(_R * _N, _D), rw, src_pre, dst_flat)
    return _combine2_call(agg2.reshape(_R, _N, _D), W2, b2, s_all)


# speedup vs baseline: 1.9891x; 1.0007x over previous
import jax
import jax.numpy as jnp
from jax import lax
from jax.experimental import pallas as pl
from jax.experimental.pallas import tpu as pltpu
from jax.experimental.pallas import tpu_sc as plsc

_N = 10000
_D = 128
_R = 4
_E = 160000

_NS = 16
_ROWS_PER_SUB = 624
_ECHUNK = 64
_NCHUNKS = _E // _ECHUNK
_DEG_CHUNK = 2000
_EPD = _E // 4

_sc_mesh = plsc.VectorSubcoreMesh(core_axis_name="c", subcore_axis_name="s")


def _degree_body(idx_hbm, out_hbm, counts_v, idx_v):
    c = lax.axis_index("c")
    s = lax.axis_index("s")
    wid = c * _NS + s
    a = wid // 4
    q = wid % 4
    zero16 = jnp.zeros((16,), jnp.float32)
    ones16 = jnp.ones((16,), jnp.float32)

    def zfill(j, carry):
        counts_v[pl.ds(j * 16, 16)] = zero16
        return carry

    lax.fori_loop(0, _N // 16, zfill, 0)

    base = a * _E + q * _EPD

    def chunk(i, carry):
        pltpu.sync_copy(idx_hbm.at[pl.ds(base + i * _DEG_CHUNK, _DEG_CHUNK)],
                        idx_v)

        def inner(j, icarry):
            v = idx_v[pl.ds(j * 16, 16)]
            plsc.addupdate_scatter(counts_v, [v], ones16)
            return icarry

        lax.fori_loop(0, _DEG_CHUNK // 16, inner, 0)
        return carry

    lax.fori_loop(0, _EPD // _DEG_CHUNK, chunk, 0)
    pltpu.sync_copy(counts_v, out_hbm.at[pl.ds(q * (8 * _N) + a * _N, _N)])


_degree_call = pl.kernel(
    _degree_body,
    out_type=jax.ShapeDtypeStruct((4 * 8 * _N,), jnp.float32),
    scratch_types=[
        pltpu.VMEM((_N,), jnp.float32),
        pltpu.VMEM((_DEG_CHUNK,), jnp.int32),
    ],
    mesh=_sc_mesh,
    compiler_params=pltpu.CompilerParams(needs_layout_passes=False),
)


_CPT = _NCHUNKS // _NS


def _edge_body(hs_hbm, rw_hbm, src_hbm, dst_hbm, out_hbm,
               agg_sp,
               rows0, rows1, rows2, rw0, rw1, rw2,
               sblk0, sblk1, dblk0, dblk1,
               sdidx0, sdidx1, sdidx2,
               lsem0, lsem1, lsem2, ssem0, ssem1, ssem2):
    c = lax.axis_index("c")
    s = lax.axis_index("s")
    zero16 = jnp.zeros((16,), jnp.float32)
    rows = (rows0, rows1, rows2)
    rwb = (rw0, rw1, rw2)
    sblk = (sblk0, sblk1)
    dblk = (dblk0, dblk1)
    sdidx = (sdidx0, sdidx1, sdidx2)
    lsem = (lsem0, lsem1, lsem2)
    ssem = (ssem0, ssem1, ssem2)
    row0 = s * _ROWS_PER_SUB
    _BLK = 6
    _NBLK = _CPT // _BLK

    def start_loads(b, p, off, ebase):
        pltpu.async_copy(hs_hbm.at[sblk[p].at[pl.ds(off, _ECHUNK)]],
                         rows[b], lsem[b])
        pltpu.async_copy(rw_hbm.at[pl.ds(ebase, _ECHUNK)], rwb[b], lsem[b])

    def wait_loads(b, p, off, ebase):
        pltpu.make_async_copy(hs_hbm.at[sblk[p].at[pl.ds(off, _ECHUNK)]],
                              rows[b], lsem[b]).wait()
        pltpu.make_async_copy(rw_hbm.at[pl.ds(ebase, _ECHUNK)], rwb[b],
                              lsem[b]).wait()

    def wait_scatter(b):
        pltpu.make_async_copy(rwb[b], agg_sp.at[sdidx[b]], ssem[b]).wait()

    def mul_into(b, p, off):
        for i8 in range(_ECHUNK // 16):
            sdidx[b][pl.ds(i8 * 16, 16)] = dblk[p][pl.ds(off + i8 * 16, 16)]

        def mul(e, icarry):
            for dcol in range(8):
                sl = pl.ds(dcol * 16, 16)
                rwb[b][e, sl] = rwb[b][e, sl] * rows[b][e, sl]
            return icarry

        lax.fori_loop(0, _ECHUNK, mul, 0)

    for j in range(2):
        r = c * 2 + j
        rel_base = r * _E

        def zfill(e, carry):
            for dcol in range(8):
                rw0[e, pl.ds(dcol * 16, 16)] = zero16
            return carry

        lax.fori_loop(0, _ECHUNK, zfill, 0)
        for kk in range(9):
            pltpu.sync_copy(rw0, agg_sp.at[pl.ds(row0 + kk * 64, 64)])

        @pl.when(s < 15)
        def _zero_tail():
            pltpu.sync_copy(rw0.at[pl.ds(0, 48)],
                            agg_sp.at[pl.ds(row0 + 576, 48)])

        @pl.when(s == 15)
        def _zero_tail15():
            pltpu.sync_copy(rw0, agg_sp.at[pl.ds(row0 + 576, 64)])

        plsc.subcore_barrier()

        chunk0 = s * _CPT
        e0 = rel_base + chunk0 * _ECHUNK
        blk_edges = _BLK * _ECHUNK
        pltpu.sync_copy(src_hbm.at[pl.ds(e0, blk_edges)], sblk0)
        pltpu.sync_copy(dst_hbm.at[pl.ds(e0, blk_edges)], dblk0)
        start_loads(0, 0, 0, e0)
        start_loads(1, 0, _ECHUNK, e0 + _ECHUNK)

        def block_pair(t, carry):
            for pb in range(2):
                g = 2 * t + pb
                gbase = e0 + g * blk_edges

                @pl.when(g < _NBLK - 1)
                def _prefetch_idx():
                    pltpu.sync_copy(
                        src_hbm.at[pl.ds(gbase + blk_edges, blk_edges)],
                        sblk[1 - pb])
                    pltpu.sync_copy(
                        dst_hbm.at[pl.ds(gbase + blk_edges, blk_edges)],
                        dblk[1 - pb])

                for tt in range(2):
                    for k in range(3):
                        off = (3 * tt + k) * _ECHUNK
                        me = gbase + off
                        wait_loads(k, pb, off, me)
                        mul_into(k, pb, off)
                        pltpu.async_copy(rwb[k], agg_sp.at[sdidx[k]],
                                         ssem[k], add=True)
                        kp = (k + 2) % 3
                        poff = off + 2 * _ECHUNK
                        pe = me + 2 * _ECHUNK
                        pp = pb if poff < blk_edges else 1 - pb
                        poff2 = poff if poff < blk_edges else poff - blk_edges
                        if tt == 0 and k == 0:
                            @pl.when(g > 0)
                            def _drain_first():
                                wait_scatter(kp)

                            start_loads(kp, pp, poff2, pe)
                        elif tt == 1 and k >= 1:
                            @pl.when(g < _NBLK - 1)
                            def _prep_tail():
                                wait_scatter(kp)
                                start_loads(kp, pp, poff2, pe)
                        else:
                            wait_scatter(kp)
                            start_loads(kp, pp, poff2, pe)
            return carry

        lax.fori_loop(0, _NBLK // 2, block_pair, 0)
        wait_scatter(0)
        wait_scatter(1)
        wait_scatter(2)

        @pl.when(s < 4)
        def _leftover():
            eL = rel_base + (_NS * _CPT + s) * _ECHUNK
            pltpu.sync_copy(src_hbm.at[pl.ds(eL, _ECHUNK)],
                            sblk0.at[pl.ds(0, _ECHUNK)])
            pltpu.sync_copy(dst_hbm.at[pl.ds(eL, _ECHUNK)],
                            dblk0.at[pl.ds(0, _ECHUNK)])
            start_loads(0, 0, 0, eL)
            wait_loads(0, 0, 0, eL)
            mul_into(0, 0, 0)
            pltpu.sync_copy(rw0, agg_sp.at[sdidx0], add=True)

        plsc.subcore_barrier()

        out_base = r * _N + row0
        for kk in range(4):
            pltpu.sync_copy(agg_sp.at[pl.ds(row0 + kk * 128, 128)],
                            out_hbm.at[pl.ds(out_base + kk * 128, 128)])

        @pl.when(s < 15)
        def _dump_tail():
            pltpu.sync_copy(agg_sp.at[pl.ds(row0 + 512, 112)],
                            out_hbm.at[pl.ds(out_base + 512, 112)])

        @pl.when(s == 15)
        def _dump_tail15():
            pltpu.sync_copy(agg_sp.at[pl.ds(row0 + 512, 128)],
                            out_hbm.at[pl.ds(out_base + 512, 128)])

        plsc.subcore_barrier()


_edge_call = pl.kernel(
    _edge_body,
    out_type=jax.ShapeDtypeStruct((4 * _N, _D), jnp.float32),
    scratch_types=(
        [pltpu.VMEM_SHARED((_N, _D), jnp.float32)]
        + [pltpu.VMEM((_ECHUNK, _D), jnp.float32)] * 6
        + [pltpu.VMEM((6 * _ECHUNK,), jnp.int32)] * 4
        + [pltpu.VMEM((_ECHUNK,), jnp.int32)] * 3
        + [pltpu.SemaphoreType.DMA] * 6
    ),
    mesh=_sc_mesh,
    compiler_params=pltpu.CompilerParams(needs_layout_passes=False),
)


_BN = 1024


def _prescale_body(feat_ref, wt_ref, b_ref, cnt_ref, hs_ref, s_ref):
    h = jnp.dot(feat_ref[...], wt_ref[...],
                preferred_element_type=jnp.float32) + b_ref[...]
    deg = jnp.sum(cnt_ref[...], axis=0)
    sfac = lax.rsqrt(jnp.maximum(deg, 1.0))
    s_ref[...] = sfac
    for r in range(_R):
        hs_ref[r] = h * sfac[r][:, None]


_prescale_call = pl.pallas_call(
    _prescale_body,
    grid=(pl.cdiv(_N, _BN),),
    in_specs=[
        pl.BlockSpec((_BN, _D), lambda i: (i, 0)),
        pl.BlockSpec((_D, _D), lambda i: (0, 0)),
        pl.BlockSpec((1, _D), lambda i: (0, 0)),
        pl.BlockSpec((4, 8, _BN), lambda i: (0, 0, i)),
    ],
    out_specs=[
        pl.BlockSpec((_R, _BN, _D), lambda i: (0, i, 0)),
        pl.BlockSpec((8, _BN), lambda i: (0, i)),
    ],
    out_shape=[
        jax.ShapeDtypeStruct((_R, _N, _D), jnp.float32),
        jax.ShapeDtypeStruct((8, _N), jnp.float32),
    ],
)

_BE = 2048


def _rw_body(ew_ref, wt_ref, b_ref, out_ref):
    out_ref[...] = jnp.dot(ew_ref[...], wt_ref[...],
                           preferred_element_type=jnp.float32) + b_ref[...]


_rw_call = pl.pallas_call(
    _rw_body,
    grid=(pl.cdiv(_R * _E, _BE),),
    in_specs=[
        pl.BlockSpec((_BE, _D), lambda i: (i, 0)),
        pl.BlockSpec((_D, _D), lambda i: (0, 0)),
        pl.BlockSpec((1, _D), lambda i: (0, 0)),
    ],
    out_specs=pl.BlockSpec((_BE, _D), lambda i: (i, 0)),
    out_shape=jax.ShapeDtypeStruct((_R * _E, _D), jnp.float32),
)


def _combine1_body(agg_ref, w_ref, b_ref, s_ref, hs2_ref):
    scaled = jnp.concatenate(
        [agg_ref[r] * s_ref[4 + r][:, None] for r in range(_R)], axis=1)
    acc = jnp.dot(scaled, w_ref[...].reshape(_R * _D, _D),
                  preferred_element_type=jnp.float32)
    emb0 = (acc + jnp.sum(b_ref[...], axis=0)[None, :]) * 0.25
    for r in range(_R):
        hs2_ref[r] = emb0 * s_ref[r][:, None]


_combine1_call = pl.pallas_call(
    _combine1_body,
    grid=(pl.cdiv(_N, _BN),),
    in_specs=[
        pl.BlockSpec((_R, _BN, _D), lambda i: (0, i, 0)),
        pl.BlockSpec((_R, _D, _D), lambda i: (0, 0, 0)),
        pl.BlockSpec((_R, _D), lambda i: (0, 0)),
        pl.BlockSpec((8, _BN), lambda i: (0, i)),
    ],
    out_specs=pl.BlockSpec((_R, _BN, _D), lambda i: (0, i, 0)),
    out_shape=jax.ShapeDtypeStruct((_R, _N, _D), jnp.float32),
)


def _combine2_body(agg_ref, w_ref, b_ref, s_ref, out_ref):
    scaled = jnp.concatenate(
        [agg_ref[r] * s_ref[4 + r][:, None] for r in range(_R)], axis=1)
    acc = jnp.dot(scaled, w_ref[...].reshape(_R * _D, _D),
                  preferred_element_type=jnp.float32)
    out_ref[...] = (acc + jnp.sum(b_ref[...], axis=0)[None, :]) * 0.25


_combine2_call = pl.pallas_call(
    _combine2_body,
    grid=(pl.cdiv(_N, _BN),),
    in_specs=[
        pl.BlockSpec((_R, _BN, _D), lambda i: (0, i, 0)),
        pl.BlockSpec((_R, _D, _D), lambda i: (0, 0, 0)),
        pl.BlockSpec((_R, _D), lambda i: (0, 0)),
        pl.BlockSpec((8, _BN), lambda i: (0, i)),
    ],
    out_specs=pl.BlockSpec((_BN, _D), lambda i: (i, 0)),
    out_shape=jax.ShapeDtypeStruct((_N, _D), jnp.float32),
)


def kernel(feat, edge_index_r0, edge_index_r1, edge_index_r2, edge_index_r3,
           edge_weight_r0, edge_weight_r1, edge_weight_r2, edge_weight_r3,
           node_fc_W, node_fc_b, rela_fc_W, rela_fc_b, W1, b1, W2, b2):
    eis = [edge_index_r0, edge_index_r1, edge_index_r2, edge_index_r3]
    src = jnp.stack([e[0] for e in eis]).astype(jnp.int32)
    dst = jnp.stack([e[1] for e in eis]).astype(jnp.int32)
    idx_all = jnp.concatenate([src, dst], axis=0).reshape(-1)

    counts = _degree_call(idx_all).reshape(4, 8, _N)
    hs, s_all = _prescale_call(feat, node_fc_W.T, node_fc_b[None, :], counts)

    ew_all = jnp.concatenate(
        [edge_weight_r0, edge_weight_r1, edge_weight_r2, edge_weight_r3], axis=0)
    rw = _rw_call(ew_all, rela_fc_W.T, rela_fc_b[None, :])

    src_pre = (src + (jnp.arange(_R, dtype=jnp.int32) * _N)[:, None]).reshape(-1)
    dst_flat = dst.reshape(-1)

    agg1 = _edge_call(hs.reshape(_R * _N, _D), rw, src_pre, dst_flat)
    hs2 = _combine1_call(agg1.reshape(_R, _N, _D), W1, b1, s_all)
    agg2 = _edge_call(hs2.reshape(_R * _N, _D), rw, src_pre, dst_flat)
    return _combine2_call(agg2.reshape(_R, _N, _D), W2, b2, s_all)


# speedup vs baseline: 1.9893x; 1.0001x over previous
import jax
import jax.numpy as jnp
from jax import lax
from jax.experimental import pallas as pl
from jax.experimental.pallas import tpu as pltpu
from jax.experimental.pallas import tpu_sc as plsc

_N = 10000
_D = 128
_R = 4
_E = 160000

_NS = 16
_ROWS_PER_SUB = 624
_ECHUNK = 64
_NCHUNKS = _E // _ECHUNK
_DEG_CHUNK = 2000
_EPD = _E // 4

_sc_mesh = plsc.VectorSubcoreMesh(core_axis_name="c", subcore_axis_name="s")


def _degree_body(idx_hbm, out_hbm, counts_v, idx_v):
    c = lax.axis_index("c")
    s = lax.axis_index("s")
    wid = c * _NS + s
    a = wid // 4
    q = wid % 4
    zero16 = jnp.zeros((16,), jnp.float32)
    ones16 = jnp.ones((16,), jnp.float32)

    def zfill(j, carry):
        counts_v[pl.ds(j * 16, 16)] = zero16
        return carry

    lax.fori_loop(0, _N // 16, zfill, 0)

    base = a * _E + q * _EPD

    def chunk(i, carry):
        pltpu.sync_copy(idx_hbm.at[pl.ds(base + i * _DEG_CHUNK, _DEG_CHUNK)],
                        idx_v)

        def inner(j, icarry):
            v = idx_v[pl.ds(j * 16, 16)]
            plsc.addupdate_scatter(counts_v, [v], ones16)
            return icarry

        lax.fori_loop(0, _DEG_CHUNK // 16, inner, 0)
        return carry

    lax.fori_loop(0, _EPD // _DEG_CHUNK, chunk, 0)
    pltpu.sync_copy(counts_v, out_hbm.at[pl.ds(q * (8 * _N) + a * _N, _N)])


_degree_call = pl.kernel(
    _degree_body,
    out_type=jax.ShapeDtypeStruct((4 * 8 * _N,), jnp.float32),
    scratch_types=[
        pltpu.VMEM((_N,), jnp.float32),
        pltpu.VMEM((_DEG_CHUNK,), jnp.int32),
    ],
    mesh=_sc_mesh,
    compiler_params=pltpu.CompilerParams(needs_layout_passes=False),
)


_CPT = _NCHUNKS // _NS


def _edge_body(hs_hbm, rw_hbm, src_hbm, dst_hbm, out_hbm,
               agg_sp,
               rows0, rows1, rows2, rw0, rw1, rw2,
               sblk0, sblk1, dblk0, dblk1,
               sdidx0, sdidx1, sdidx2,
               lsem0, lsem1, lsem2, ssem0, ssem1, ssem2):
    c = lax.axis_index("c")
    s = lax.axis_index("s")
    zero16 = jnp.zeros((16,), jnp.float32)
    rows = (rows0, rows1, rows2)
    rwb = (rw0, rw1, rw2)
    sblk = (sblk0, sblk1)
    dblk = (dblk0, dblk1)
    sdidx = (sdidx0, sdidx1, sdidx2)
    lsem = (lsem0, lsem1, lsem2)
    ssem = (ssem0, ssem1, ssem2)
    row0 = s * _ROWS_PER_SUB
    _BLK = 6
    _NBLK = _CPT // _BLK

    def start_loads(b, p, off, ebase):
        pltpu.async_copy(hs_hbm.at[sblk[p].at[pl.ds(off, _ECHUNK)]],
                         rows[b], lsem[b])
        pltpu.async_copy(rw_hbm.at[pl.ds(ebase, _ECHUNK)], rwb[b], lsem[b])

    def wait_loads(b, p, off, ebase):
        pltpu.make_async_copy(hs_hbm.at[sblk[p].at[pl.ds(off, _ECHUNK)]],
                              rows[b], lsem[b]).wait()
        pltpu.make_async_copy(rw_hbm.at[pl.ds(ebase, _ECHUNK)], rwb[b],
                              lsem[b]).wait()

    def wait_scatter(b):
        pltpu.make_async_copy(rwb[b], agg_sp.at[sdidx[b]], ssem[b]).wait()

    def mul_into(b, p, off):
        for i8 in range(_ECHUNK // 16):
            sdidx[b][pl.ds(i8 * 16, 16)] = dblk[p][pl.ds(off + i8 * 16, 16)]

        def mul(e, icarry):
            for dcol in range(8):
                sl = pl.ds(dcol * 16, 16)
                rwb[b][e, sl] = rwb[b][e, sl] * rows[b][e, sl]
            return icarry

        lax.fori_loop(0, _ECHUNK, mul, 0)

    for j in range(2):
        r = c * 2 + j
        rel_base = r * _E

        def zfill(e, carry):
            for dcol in range(8):
                rw0[e, pl.ds(dcol * 16, 16)] = zero16
            return carry

        lax.fori_loop(0, _ECHUNK, zfill, 0)
        for kk in range(9):
            pltpu.sync_copy(rw0, agg_sp.at[pl.ds(row0 + kk * 64, 64)])

        @pl.when(s < 15)
        def _zero_tail():
            pltpu.sync_copy(rw0.at[pl.ds(0, 48)],
                            agg_sp.at[pl.ds(row0 + 576, 48)])

        @pl.when(s == 15)
        def _zero_tail15():
            pltpu.sync_copy(rw0, agg_sp.at[pl.ds(row0 + 576, 64)])

        plsc.subcore_barrier()

        chunk0 = s * _CPT
        e0 = rel_base + chunk0 * _ECHUNK
        blk_edges = _BLK * _ECHUNK
        pltpu.sync_copy(src_hbm.at[pl.ds(e0, blk_edges)], sblk0)
        pltpu.sync_copy(dst_hbm.at[pl.ds(e0, blk_edges)], dblk0)
        start_loads(0, 0, 0, e0)
        start_loads(1, 0, _ECHUNK, e0 + _ECHUNK)

        def block_pair(t, carry):
            for pb in range(2):
                g = 2 * t + pb
                gbase = e0 + g * blk_edges

                @pl.when(g < _NBLK - 1)
                def _prefetch_idx():
                    pltpu.sync_copy(
                        src_hbm.at[pl.ds(gbase + blk_edges, blk_edges)],
                        sblk[1 - pb])
                    pltpu.sync_copy(
                        dst_hbm.at[pl.ds(gbase + blk_edges, blk_edges)],
                        dblk[1 - pb])

                for tt in range(2):
                    for k in range(3):
                        off = (3 * tt + k) * _ECHUNK
                        me = gbase + off
                        wait_loads(k, pb, off, me)
                        mul_into(k, pb, off)
                        pltpu.async_copy(rwb[k], agg_sp.at[sdidx[k]],
                                         ssem[k], add=True)
                        kp = (k + 2) % 3
                        poff = off + 2 * _ECHUNK
                        pe = me + 2 * _ECHUNK
                        pp = pb if poff < blk_edges else 1 - pb
                        poff2 = poff if poff < blk_edges else poff - blk_edges
                        if tt == 0 and k == 0:
                            @pl.when(g > 0)
                            def _drain_first():
                                wait_scatter(kp)

                            start_loads(kp, pp, poff2, pe)
                        elif tt == 1 and k >= 1:
                            @pl.when(g < _NBLK - 1)
                            def _prep_tail():
                                wait_scatter(kp)
                                start_loads(kp, pp, poff2, pe)
                        else:
                            wait_scatter(kp)
                            start_loads(kp, pp, poff2, pe)
            return carry

        lax.fori_loop(0, _NBLK // 2, block_pair, 0)
        wait_scatter(0)
        wait_scatter(1)
        wait_scatter(2)

        @pl.when(s < 4)
        def _leftover():
            eL = rel_base + (_NS * _CPT + s) * _ECHUNK
            pltpu.sync_copy(src_hbm.at[pl.ds(eL, _ECHUNK)],
                            sblk0.at[pl.ds(0, _ECHUNK)])
            pltpu.sync_copy(dst_hbm.at[pl.ds(eL, _ECHUNK)],
                            dblk0.at[pl.ds(0, _ECHUNK)])
            start_loads(0, 0, 0, eL)
            wait_loads(0, 0, 0, eL)
            mul_into(0, 0, 0)
            pltpu.sync_copy(rw0, agg_sp.at[sdidx0], add=True)

        plsc.subcore_barrier()

        out_base = r * _N + row0
        for kk in range(4):
            pltpu.sync_copy(agg_sp.at[pl.ds(row0 + kk * 128, 128)],
                            out_hbm.at[pl.ds(out_base + kk * 128, 128)])

        @pl.when(s < 15)
        def _dump_tail():
            pltpu.sync_copy(agg_sp.at[pl.ds(row0 + 512, 112)],
                            out_hbm.at[pl.ds(out_base + 512, 112)])

        @pl.when(s == 15)
        def _dump_tail15():
            pltpu.sync_copy(agg_sp.at[pl.ds(row0 + 512, 128)],
                            out_hbm.at[pl.ds(out_base + 512, 128)])

        plsc.subcore_barrier()


_edge_call = pl.kernel(
    _edge_body,
    out_type=jax.ShapeDtypeStruct((4 * _N, _D), jnp.float32),
    scratch_types=(
        [pltpu.VMEM_SHARED((_N, _D), jnp.float32)]
        + [pltpu.VMEM((_ECHUNK, _D), jnp.float32)] * 6
        + [pltpu.VMEM((6 * _ECHUNK,), jnp.int32)] * 4
        + [pltpu.VMEM((_ECHUNK,), jnp.int32)] * 3
        + [pltpu.SemaphoreType.DMA] * 6
    ),
    mesh=_sc_mesh,
    compiler_params=pltpu.CompilerParams(needs_layout_passes=False),
)


_BN = 1024


def _prescale_body(feat_ref, wt_ref, b_ref, cnt_ref, hs_ref, s_ref):
    h = jnp.dot(feat_ref[...], wt_ref[...],
                preferred_element_type=jnp.float32) + b_ref[...]
    deg = jnp.sum(cnt_ref[...], axis=0)
    sfac = lax.rsqrt(jnp.maximum(deg, 1.0))
    s_ref[...] = sfac
    for r in range(_R):
        hs_ref[r] = h * sfac[r][:, None]


_prescale_call = pl.pallas_call(
    _prescale_body,
    grid=(pl.cdiv(_N, _BN),),
    in_specs=[
        pl.BlockSpec((_BN, _D), lambda i: (i, 0)),
        pl.BlockSpec((_D, _D), lambda i: (0, 0)),
        pl.BlockSpec((1, _D), lambda i: (0, 0)),
        pl.BlockSpec((4, 8, _BN), lambda i: (0, 0, i)),
    ],
    out_specs=[
        pl.BlockSpec((_R, _BN, _D), lambda i: (0, i, 0)),
        pl.BlockSpec((8, _BN), lambda i: (0, i)),
    ],
    out_shape=[
        jax.ShapeDtypeStruct((_R, _N, _D), jnp.float32),
        jax.ShapeDtypeStruct((8, _N), jnp.float32),
    ],
)

_BE = 2048


def _rw_body(ew_ref, wt_ref, b_ref, out_ref):
    out_ref[...] = jnp.dot(ew_ref[...], wt_ref[...],
                           preferred_element_type=jnp.float32) + b_ref[...]


_rw_call = pl.pallas_call(
    _rw_body,
    grid=(pl.cdiv(_R * _E, _BE),),
    in_specs=[
        pl.BlockSpec((_BE, _D), lambda i: (i, 0)),
        pl.BlockSpec((_D, _D), lambda i: (0, 0)),
        pl.BlockSpec((1, _D), lambda i: (0, 0)),
    ],
    out_specs=pl.BlockSpec((_BE, _D), lambda i: (i, 0)),
    out_shape=jax.ShapeDtypeStruct((_R * _E, _D), jnp.float32),
)


def _combine1_body(agg_ref, w_ref, b_ref, s_ref, hs2_ref):
    acc = None
    for r in range(_R):
        t = jnp.dot(agg_ref[r], w_ref[r], preferred_element_type=jnp.float32)
        t = t * s_ref[4 + r][:, None] + b_ref[r][None, :]
        acc = t if acc is None else acc + t
    emb0 = acc * 0.25
    for r in range(_R):
        hs2_ref[r] = emb0 * s_ref[r][:, None]


_combine1_call = pl.pallas_call(
    _combine1_body,
    grid=(pl.cdiv(_N, _BN),),
    in_specs=[
        pl.BlockSpec((_R, _BN, _D), lambda i: (0, i, 0)),
        pl.BlockSpec((_R, _D, _D), lambda i: (0, 0, 0)),
        pl.BlockSpec((_R, _D), lambda i: (0, 0)),
        pl.BlockSpec((8, _BN), lambda i: (0, i)),
    ],
    out_specs=pl.BlockSpec((_R, _BN, _D), lambda i: (0, i, 0)),
    out_shape=jax.ShapeDtypeStruct((_R, _N, _D), jnp.float32),
)


def _combine2_body(agg_ref, w_ref, b_ref, s_ref, out_ref):
    acc = None
    for r in range(_R):
        t = jnp.dot(agg_ref[r], w_ref[r], preferred_element_type=jnp.float32)
        t = t * s_ref[4 + r][:, None] + b_ref[r][None, :]
        acc = t if acc is None else acc + t
    out_ref[...] = acc * 0.25


_combine2_call = pl.pallas_call(
    _combine2_body,
    grid=(pl.cdiv(_N, _BN),),
    in_specs=[
        pl.BlockSpec((_R, _BN, _D), lambda i: (0, i, 0)),
        pl.BlockSpec((_R, _D, _D), lambda i: (0, 0, 0)),
        pl.BlockSpec((_R, _D), lambda i: (0, 0)),
        pl.BlockSpec((8, _BN), lambda i: (0, i)),
    ],
    out_specs=pl.BlockSpec((_BN, _D), lambda i: (i, 0)),
    out_shape=jax.ShapeDtypeStruct((_N, _D), jnp.float32),
)


def kernel(feat, edge_index_r0, edge_index_r1, edge_index_r2, edge_index_r3,
           edge_weight_r0, edge_weight_r1, edge_weight_r2, edge_weight_r3,
           node_fc_W, node_fc_b, rela_fc_W, rela_fc_b, W1, b1, W2, b2):
    eis = [edge_index_r0, edge_index_r1, edge_index_r2, edge_index_r3]
    src = jnp.stack([e[0] for e in eis]).astype(jnp.int32)
    dst = jnp.stack([e[1] for e in eis]).astype(jnp.int32)
    idx_all = jnp.concatenate([src, dst], axis=0).reshape(-1)

    counts = _degree_call(idx_all).reshape(4, 8, _N)
    hs, s_all = _prescale_call(feat, node_fc_W.T, node_fc_b[None, :], counts)

    ew_all = jnp.concatenate(
        [edge_weight_r0, edge_weight_r1, edge_weight_r2, edge_weight_r3], axis=0)
    rw = _rw_call(ew_all, rela_fc_W.T, rela_fc_b[None, :])

    src_pre = (src + (jnp.arange(_R, dtype=jnp.int32) * _N)[:, None]).reshape(-1)
    dst_flat = dst.reshape(-1)

    agg1 = _edge_call(hs.reshape(_R * _N, _D), rw, src_pre, dst_flat)
    hs2 = _combine1_call(agg1.reshape(_R, _N, _D), W1, b1, s_all)
    agg2 = _edge_call(hs2.reshape(_R * _N, _D), rw, src_pre, dst_flat)
    return _combine2_call(agg2.reshape(_R, _N, _D), W2, b2, s_all)
